# Initial kernel scaffold; baseline (speedup 1.0000x reference)
import functools

import jax
import jax.numpy as jnp
from jax import lax
from jax.experimental import pallas as pl
from jax.experimental.pallas import tpu as pltpu
from jax.experimental.pallas import tpu_sc as plsc

N_NODES = 10000
D_IN = 128
D_AUG = 144
N_EDGES = 320000
NUM_CORES = 2
NUM_TILES = 16
CHUNK = 128
K_CHUNKS = 157
PT = K_CHUNKS * CHUNK
E_PAD = NUM_TILES * PT
ROWS_PER_TILE = N_NODES // NUM_TILES


def _sc_aggregate(feat_aug, gidx, sidx, wts):
  mesh = plsc.VectorSubcoreMesh(core_axis_name="c", subcore_axis_name="s")

  @functools.partial(
      pl.kernel,
      mesh=mesh,
      out_type=jax.ShapeDtypeStruct((NUM_CORES, N_NODES, D_AUG), jnp.float32),
      scratch_types=[
          pltpu.VMEM((K_CHUNKS, CHUNK), jnp.int32),
          pltpu.VMEM((K_CHUNKS, CHUNK), jnp.int32),
          pltpu.VMEM((PT,), jnp.float32),
          pltpu.VMEM((CHUNK, D_AUG), jnp.float32),
          pltpu.VMEM_SHARED((N_NODES, D_AUG), jnp.float32),
      ],
  )
  def k(feat_h, gidx_h, sidx_h, wts_h, out_h, gidx_v, sidx_v, w_v, rows_v,
        acc):
    c = lax.axis_index("c")
    s = lax.axis_index("s")
    base = s * ROWS_PER_TILE

    pltpu.sync_copy(gidx_h.at[c, s], gidx_v)
    pltpu.sync_copy(sidx_h.at[c, s], sidx_v)
    pltpu.sync_copy(wts_h.at[s], w_v)

    def zrow(i, carry):
      for d in range(D_AUG // 16):
        rows_v[i, pl.ds(d * 16, 16)] = jnp.zeros((16,), jnp.float32)
      return carry
    lax.fori_loop(0, CHUNK, zrow, 0)
    for t in range(ROWS_PER_TILE // CHUNK):
      pltpu.sync_copy(rows_v, acc.at[pl.ds(base + t * CHUNK, CHUNK)])
    rem = ROWS_PER_TILE % CHUNK
    if rem:
      pltpu.sync_copy(rows_v.at[pl.ds(0, rem)],
                      acc.at[pl.ds(base + ROWS_PER_TILE - rem, rem)])
    plsc.subcore_barrier()

    def body(j, carry):
      pltpu.sync_copy(feat_h.at[gidx_v.at[j]], rows_v)

      def mul(e, carry2):
        wsc = w_v[j * CHUNK + e]
        for d in range(D_AUG // 16):
          sl = pl.ds(d * 16, 16)
          rows_v[e, sl] = rows_v[e, sl] * wsc
        return carry2
      lax.fori_loop(0, CHUNK, mul, 0)

      pltpu.sync_copy(rows_v, acc.at[sidx_v.at[j]], add=True)
      return carry
    lax.fori_loop(0, K_CHUNKS, body, 0)
    plsc.subcore_barrier()

    pltpu.sync_copy(acc.at[pl.ds(base, ROWS_PER_TILE)],
                    out_h.at[c, pl.ds(base, ROWS_PER_TILE)])

  return k(feat_aug, gidx, sidx, wts)


def _tc_dense(agg, feat, w1t, w2t, a1, a2, whht, bih, bhh):
  blk = 1000
  grid = (N_NODES // blk,)

  def body(agg_ref, feat_ref, w1_ref, w2_ref, a1_ref, a2_ref, whh_ref,
           bih_ref, bhh_ref, out_ref):
    m1 = agg_ref[0]
    m2 = agg_ref[1]
    ws1 = m1[:, D_IN:D_IN + 1]
    ws2 = m2[:, D_IN:D_IN + 1]
    neigh1 = jnp.where(ws1 > 0, m1[:, :D_IN] / jnp.where(ws1 > 0, ws1, 1.0),
                       0.0)
    neigh2 = jnp.where(ws2 > 0, m2[:, :D_IN] / jnp.where(ws2 > 0, ws2, 1.0),
                       0.0)
    dot = functools.partial(jnp.dot, precision=lax.Precision.HIGHEST,
                            preferred_element_type=jnp.float32)
    n1 = dot(neigh1, w1_ref[...])
    n2 = dot(neigh2, w2_ref[...])
    gi = dot(n1, a1_ref[...]) + dot(n2, a2_ref[...]) + bih_ref[...]
    ft = feat_ref[...]
    gh = dot(ft, whh_ref[...]) + bhh_ref[...]
    r = jax.nn.sigmoid(gi[:, :D_IN] + gh[:, :D_IN])
    z = jax.nn.sigmoid(gi[:, D_IN:2 * D_IN] + gh[:, D_IN:2 * D_IN])
    n = jnp.tanh(gi[:, 2 * D_IN:] + r * gh[:, 2 * D_IN:])
    out_ref[...] = (1.0 - z) * n + z * ft

  return pl.pallas_call(
      body,
      grid=grid,
      in_specs=[
          pl.BlockSpec((NUM_CORES, blk, D_AUG), lambda i: (0, i, 0)),
          pl.BlockSpec((blk, D_IN), lambda i: (i, 0)),
          pl.BlockSpec((D_IN, D_IN), lambda i: (0, 0)),
          pl.BlockSpec((D_IN, D_IN), lambda i: (0, 0)),
          pl.BlockSpec((D_IN, 3 * D_IN), lambda i: (0, 0)),
          pl.BlockSpec((D_IN, 3 * D_IN), lambda i: (0, 0)),
          pl.BlockSpec((D_IN, 3 * D_IN), lambda i: (0, 0)),
          pl.BlockSpec((1, 3 * D_IN), lambda i: (0, 0)),
          pl.BlockSpec((1, 3 * D_IN), lambda i: (0, 0)),
      ],
      out_specs=pl.BlockSpec((blk, D_IN), lambda i: (i, 0)),
      out_shape=jax.ShapeDtypeStruct((N_NODES, D_IN), jnp.float32),
  )(agg, feat, w1t, w2t, a1, a2, whht, bih, bhh)


@jax.jit
def kernel(feat, edge_index, edge_weight, W1, W2, W_ih, W_hh, b_ih, b_hh):
  pad = E_PAD - N_EDGES
  src = jnp.concatenate([edge_index[0], jnp.zeros((pad,), jnp.int32)])
  dst = jnp.concatenate([edge_index[1], jnp.zeros((pad,), jnp.int32)])
  w = jnp.concatenate([edge_weight, jnp.zeros((pad,), jnp.float32)])
  gidx = jnp.stack([src, dst]).reshape(2, NUM_TILES, K_CHUNKS, CHUNK)
  sidx = jnp.stack([dst, src]).reshape(2, NUM_TILES, K_CHUNKS, CHUNK)
  wts = w.reshape(NUM_TILES, PT)
  feat_aug = jnp.concatenate(
      [feat, jnp.ones((N_NODES, 1), jnp.float32),
       jnp.zeros((N_NODES, D_AUG - D_IN - 1), jnp.float32)], axis=1)

  agg = _sc_aggregate(feat_aug, gidx, sidx, wts)

  w1t = W1.T
  w2t = W2.T
  wiht = W_ih.T
  a1 = wiht[:D_IN]
  a2 = wiht[D_IN:]
  whht = W_hh.T
  bih = b_ih.reshape(1, 3 * D_IN)
  bhh = b_hh.reshape(1, 3 * D_IN)
  return _tc_dense(agg, feat, w1t, w2t, a1, a2, whht, bih, bhh)


# speedup vs baseline: 3.4190x; 3.4190x over previous
import functools

import jax
import jax.numpy as jnp
from jax import lax
from jax.experimental import pallas as pl
from jax.experimental.pallas import tpu as pltpu
from jax.experimental.pallas import tpu_sc as plsc

N_NODES = 10000
D_IN = 128
D_AUG = 144
N_EDGES = 320000
NUM_CORES = 2
NUM_TILES = 16
CHUNK = 128
K_CHUNKS = 160
SB = 16
NB = K_CHUNKS // SB
PT = K_CHUNKS * CHUNK
E_PAD = NUM_TILES * PT
N_PAD = 10240
ROWS_PER_TILE = N_PAD // NUM_TILES


def _sc_aggregate(feat_aug, gidx, sidx, wts):
  mesh = plsc.VectorSubcoreMesh(core_axis_name="c", subcore_axis_name="s")

  @functools.partial(
      pl.kernel,
      mesh=mesh,
      compiler_params=pltpu.CompilerParams(use_tc_tiling_on_sc=False),
      out_type=jax.ShapeDtypeStruct((NUM_CORES, N_PAD, D_AUG), jnp.float32),
      scratch_types=[
          pltpu.VMEM((SB, CHUNK), jnp.int32),
          pltpu.VMEM((SB, CHUNK), jnp.int32),
          pltpu.VMEM((SB * CHUNK,), jnp.float32),
          pltpu.VMEM((CHUNK, D_AUG), jnp.float32),
          pltpu.VMEM_SHARED((N_PAD, D_AUG), jnp.float32),
      ],
  )
  def k(feat_h, gidx_h, sidx_h, wts_h, out_h, gidx_v, sidx_v, w_v, rows_v,
        acc):
    c = lax.axis_index("c")
    s = lax.axis_index("s")
    base = s * ROWS_PER_TILE

    def zrow(i, carry):
      for d in range(D_AUG // 16):
        rows_v[i, pl.ds(d * 16, 16)] = jnp.zeros((16,), jnp.float32)
      return carry
    lax.fori_loop(0, CHUNK, zrow, 0)
    for t in range(ROWS_PER_TILE // CHUNK):
      pltpu.sync_copy(rows_v, acc.at[pl.ds(base + t * CHUNK, CHUNK)])
    rem = ROWS_PER_TILE % CHUNK
    if rem:
      pltpu.sync_copy(rows_v.at[pl.ds(0, rem)],
                      acc.at[pl.ds(base + ROWS_PER_TILE - rem, rem)])
    plsc.subcore_barrier()

    def blk(b, carry):
      pltpu.sync_copy(gidx_h.at[c, s, pl.ds(b * SB, SB)], gidx_v)
      pltpu.sync_copy(sidx_h.at[c, s, pl.ds(b * SB, SB)], sidx_v)
      pltpu.sync_copy(wts_h.at[s, pl.ds(b * SB * CHUNK, SB * CHUNK)], w_v)

      def body(j, carry1):
        pltpu.sync_copy(feat_h.at[gidx_v.at[j]], rows_v)

        def mul(g, carry2):
          wv16 = w_v[pl.ds(j * CHUNK + g * 16, 16)]
          for e16 in range(16):
            wsc = wv16[e16]
            row = g * 16 + e16
            for d in range(D_AUG // 16):
              sl = pl.ds(d * 16, 16)
              rows_v[row, sl] = rows_v[row, sl] * wsc
          return carry2
        lax.fori_loop(0, CHUNK // 16, mul, 0)

        pltpu.sync_copy(rows_v, acc.at[sidx_v.at[j]], add=True)
        return carry1
      lax.fori_loop(0, SB, body, 0)
      return carry
    lax.fori_loop(0, NB, blk, 0)
    plsc.subcore_barrier()

    pltpu.sync_copy(acc.at[pl.ds(base, ROWS_PER_TILE)],
                    out_h.at[c, pl.ds(base, ROWS_PER_TILE)])

  return k(feat_aug, gidx, sidx, wts)


def _tc_dense(agg, feat, w1t, w2t, a1, a2, whht, bih, bhh):
  blk = 1000
  grid = (N_NODES // blk,)

  def body(agg_ref, feat_ref, w1_ref, w2_ref, a1_ref, a2_ref, whh_ref,
           bih_ref, bhh_ref, out_ref):
    m1 = agg_ref[0]
    m2 = agg_ref[1]
    ws1 = m1[:, D_IN:D_IN + 1]
    ws2 = m2[:, D_IN:D_IN + 1]
    neigh1 = jnp.where(ws1 > 0, m1[:, :D_IN] / jnp.where(ws1 > 0, ws1, 1.0),
                       0.0)
    neigh2 = jnp.where(ws2 > 0, m2[:, :D_IN] / jnp.where(ws2 > 0, ws2, 1.0),
                       0.0)
    dot = functools.partial(jnp.dot, precision=lax.Precision.HIGHEST,
                            preferred_element_type=jnp.float32)
    n1 = dot(neigh1, w1_ref[...])
    n2 = dot(neigh2, w2_ref[...])
    gi = dot(n1, a1_ref[...]) + dot(n2, a2_ref[...]) + bih_ref[...]
    ft = feat_ref[...]
    gh = dot(ft, whh_ref[...]) + bhh_ref[...]
    r = jax.nn.sigmoid(gi[:, :D_IN] + gh[:, :D_IN])
    z = jax.nn.sigmoid(gi[:, D_IN:2 * D_IN] + gh[:, D_IN:2 * D_IN])
    n = jnp.tanh(gi[:, 2 * D_IN:] + r * gh[:, 2 * D_IN:])
    out_ref[...] = (1.0 - z) * n + z * ft

  return pl.pallas_call(
      body,
      grid=grid,
      in_specs=[
          pl.BlockSpec((NUM_CORES, blk, D_AUG), lambda i: (0, i, 0)),
          pl.BlockSpec((blk, D_IN), lambda i: (i, 0)),
          pl.BlockSpec((D_IN, D_IN), lambda i: (0, 0)),
          pl.BlockSpec((D_IN, D_IN), lambda i: (0, 0)),
          pl.BlockSpec((D_IN, 3 * D_IN), lambda i: (0, 0)),
          pl.BlockSpec((D_IN, 3 * D_IN), lambda i: (0, 0)),
          pl.BlockSpec((D_IN, 3 * D_IN), lambda i: (0, 0)),
          pl.BlockSpec((1, 3 * D_IN), lambda i: (0, 0)),
          pl.BlockSpec((1, 3 * D_IN), lambda i: (0, 0)),
      ],
      out_specs=pl.BlockSpec((blk, D_IN), lambda i: (i, 0)),
      out_shape=jax.ShapeDtypeStruct((N_NODES, D_IN), jnp.float32),
  )(agg, feat, w1t, w2t, a1, a2, whht, bih, bhh)


@jax.jit
def kernel(feat, edge_index, edge_weight, W1, W2, W_ih, W_hh, b_ih, b_hh):
  pad = E_PAD - N_EDGES
  src = jnp.concatenate([edge_index[0], jnp.zeros((pad,), jnp.int32)])
  dst = jnp.concatenate([edge_index[1], jnp.zeros((pad,), jnp.int32)])
  w = jnp.concatenate([edge_weight, jnp.zeros((pad,), jnp.float32)])
  gidx = jnp.stack([src, dst]).reshape(2, NUM_TILES, K_CHUNKS, CHUNK)
  sidx = jnp.stack([dst, src]).reshape(2, NUM_TILES, K_CHUNKS, CHUNK)
  wts = w.reshape(NUM_TILES, PT)
  feat_aug = jnp.concatenate(
      [feat, jnp.ones((N_NODES, 1), jnp.float32),
       jnp.zeros((N_NODES, D_AUG - D_IN - 1), jnp.float32)], axis=1)

  feat_aug, gidx, sidx, wts = lax.optimization_barrier(
      (feat_aug, gidx, sidx, wts))
  agg = _sc_aggregate(feat_aug, gidx, sidx, wts)

  w1t = W1.T
  w2t = W2.T
  wiht = W_ih.T
  a1 = wiht[:D_IN]
  a2 = wiht[D_IN:]
  whht = W_hh.T
  bih = b_ih.reshape(1, 3 * D_IN)
  bhh = b_hh.reshape(1, 3 * D_IN)
  return _tc_dense(agg, feat, w1t, w2t, a1, a2, whht, bih, bhh)
